# baseline (speedup 1.0000x reference)
import functools
import jax
import jax.numpy as jnp
from jax import lax
from jax.experimental import pallas as pl
from jax.experimental.pallas import tpu as pltpu
from jax.experimental.pallas import tpu_sc as plsc

_D = 256
_HW = 4 * _D
_N_DRUG = 1373
_N_HETE = 1546
_B = 16384
_NC = 2
_NS = 16
_NW = _NC * _NS
_CB = 16
_RPC = 2 * _CB
_BPW = _B // _NW
_NCHUNK = _BPW // _CB
_L = 16


def _body(tab, idxi, out, inter, bigA, bigB,
          gsA, gsB, wsA, wsB):
    wid = lax.axis_index("s") * _NC + lax.axis_index("c")
    base_w = wid * _BPW
    bufs = (bigA, bigB)
    gsems = (gsA, gsB)
    wsems = (wsA, wsB)

    pltpu.sync_copy(idxi.at[pl.ds(2 * base_w, 2 * _BPW)], inter)

    def fire_gather(c, buf, sem):
        pltpu.async_copy(tab.at[inter.at[pl.ds(c * _RPC, _RPC)]], buf, sem)

    def drain_gather(buf, sem):
        pltpu.make_async_copy(out.at[pl.ds(0, _RPC), :], buf, sem).wait()

    def fire_write(c, buf, sem):
        pltpu.async_copy(
            buf, out.at[pl.ds(2 * base_w + c * _RPC, _RPC), :], sem)

    def drain_write(buf, sem):
        pltpu.make_async_copy(out.at[pl.ds(0, _RPC), :], buf, sem).wait()

    fire_gather(0, bufs[0], gsems[0])
    drain_gather(bufs[0], gsems[0])
    fire_write(0, bufs[0], wsems[0])
    fire_gather(1, bufs[1], gsems[1])

    def outer(o, _):
        for step in range(2):
            c = 2 * o + 1 + step
            x = (1 + step) % 2
            y = 1 - x
            drain_gather(bufs[x], gsems[x])
            fire_write(c, bufs[x], wsems[x])
            drain_write(bufs[y], wsems[y])
            fire_gather(c + 1, bufs[y], gsems[y])
        return ()

    lax.fori_loop(0, (_NCHUNK - 2) // 2, outer, (), unroll=False)

    cl = _NCHUNK - 1
    xl = cl % 2
    yl = 1 - xl
    drain_gather(bufs[xl], gsems[xl])
    fire_write(cl, bufs[xl], wsems[xl])
    drain_write(bufs[yl], wsems[yl])
    drain_write(bufs[xl], wsems[xl])


@jax.jit
def _run(h1, h2, d1, d2, m1, m2, x1, x2):
    tab = jnp.concatenate([
        jnp.concatenate([d1, d2, h1[:_N_DRUG], h2[:_N_DRUG]], axis=1),
        jnp.concatenate([m1, m2, h1[_N_DRUG:], h2[_N_DRUG:]], axis=1),
    ], axis=0)
    mesh = plsc.VectorSubcoreMesh(core_axis_name="c", subcore_axis_name="s")
    f = pl.kernel(
        _body,
        out_type=jax.ShapeDtypeStruct((2 * _B, _HW), jnp.float32),
        mesh=mesh,
        scratch_types=[
            pltpu.VMEM((2 * _BPW,), jnp.int32),
            pltpu.VMEM((_RPC, _HW), jnp.float32),
            pltpu.VMEM((_RPC, _HW), jnp.float32),
            pltpu.SemaphoreType.DMA,
            pltpu.SemaphoreType.DMA,
            pltpu.SemaphoreType.DMA,
            pltpu.SemaphoreType.DMA,
        ],
    )
    idxi = jnp.stack([x1, x2 + _N_DRUG], axis=1).reshape(-1)
    return f(tab, idxi)


def kernel(hete_1hop, hete_2hop, drug_homo_1hop, drug_homo_2hop,
           mic_homo_1hop, mic_homo_2hop, x1, x2):
    out = _run(hete_1hop, hete_2hop, drug_homo_1hop, drug_homo_2hop,
               mic_homo_1hop, mic_homo_2hop,
               x1.astype(jnp.int32), x2.astype(jnp.int32))
    return out.reshape(_B, 1, 2, _HW)
